# baseline (speedup 1.0000x reference)
import jax
import jax.numpy as jnp
from jax.experimental import pallas as pl


def _vq_body(z_ref, cb_ref, o_ref):
    zb = z_ref[0]
    cb = cb_ref[...]
    csqr = jnp.sum(cb * cb, axis=1, keepdims=True)
    xsqr = jnp.sum(zb * zb, axis=0, keepdims=True)
    m = jax.lax.dot_general(cb, zb, (((1,), (0,)), ((), ())),
                            preferred_element_type=jnp.float32)
    dist = (csqr + xsqr) - 2.0 * m
    mn = jnp.min(dist, axis=0, keepdims=True)
    ids = jax.lax.broadcasted_iota(jnp.int32, dist.shape, 0)
    k = dist.shape[0]
    idx = jnp.min(jnp.where(dist == mn, ids, k), axis=0)
    o_ref[0, 0, :] = idx.astype(jnp.int32)


def kernel(z_e_x, codebook):
    b, d, h, w = z_e_x.shape
    t = h * w
    k = codebook.shape[0]
    z = z_e_x.reshape(b, d, t)
    out = pl.pallas_call(
        _vq_body,
        grid=(b,),
        in_specs=[
            pl.BlockSpec((1, d, t), lambda i: (i, 0, 0)),
            pl.BlockSpec((k, d), lambda i: (0, 0)),
        ],
        out_specs=pl.BlockSpec((1, 1, t), lambda i: (i, 0, 0)),
        out_shape=jax.ShapeDtypeStruct((b, 1, t), jnp.int32),
    )(z, codebook)
    return out.reshape(b, h, w)
